# baseline (speedup 1.0000x reference)
import jax
import jax.numpy as jnp
from jax.experimental import pallas as pl

_K = 1024
_D = 64


def _vq_tc_body(x_ref, cb_ref, idx_ref, zq_ref):
    xb = x_ref[0]
    cb = cb_ref[...]
    cbn = cb * -2.0
    sneg = jax.lax.dot_general(cbn, xb, (((1,), (0,)), ((), ())),
                               preferred_element_type=jnp.float32)
    c2 = 0.25 * jnp.sum(cbn * cbn, axis=1, keepdims=True)
    x2 = jnp.sum(xb * xb, axis=0, keepdims=True)
    d2 = (x2 + c2) + sneg
    e2 = jnp.maximum(d2, 1e-30)
    dist = jax.lax.rsqrt(e2) * e2
    mn = jnp.min(dist, axis=0, keepdims=True)
    kio = jax.lax.broadcasted_iota(jnp.int32, (_K, 1), 0)
    idx = jnp.min(jnp.where(dist == mn, kio, jnp.int32(2**30)), axis=0)
    idx_ref[0, 0] = idx
    onehot = (kio == idx[None, :]).astype(jnp.float32)
    z_t = -0.5 * jax.lax.dot_general(cbn, onehot, (((0,), (0,)), ((), ())),
                                     preferred_element_type=jnp.float32)
    zq_ref[0] = xb + (z_t - xb)


def kernel(x, codebook):
    b, c, l = x.shape
    idx3, zq = pl.pallas_call(
        _vq_tc_body,
        grid=(b,),
        in_specs=[
            pl.BlockSpec((1, c, l), lambda i: (i, 0, 0)),
            pl.BlockSpec((_K, _D), lambda i: (0, 0)),
        ],
        out_specs=[
            pl.BlockSpec((1, 1, l), lambda i: (i, 0, 0)),
            pl.BlockSpec((1, c, l), lambda i: (i, 0, 0)),
        ],
        out_shape=[
            jax.ShapeDtypeStruct((b, 1, l), jnp.int32),
            jax.ShapeDtypeStruct((b, c, l), jnp.float32),
        ],
    )(x, codebook)
    return (zq, x, idx3.reshape(b, l))


# speedup vs baseline: 1.0760x; 1.0760x over previous
import jax
import jax.numpy as jnp
from jax.experimental import pallas as pl

_K = 1024
_D = 64


def _vq_tc_body(x_ref, cb_ref, idx_ref, zq_ref, xo_ref):
    xb = x_ref[0]
    cb = cb_ref[...]
    cbn = cb * -2.0
    sneg = jax.lax.dot_general(cbn, xb, (((1,), (0,)), ((), ())),
                               preferred_element_type=jnp.float32)
    c2 = 0.25 * jnp.sum(cbn * cbn, axis=1, keepdims=True)
    x2 = jnp.sum(xb * xb, axis=0, keepdims=True)
    d2 = (x2 + c2) + sneg
    e2 = jnp.maximum(d2, 1e-30)
    dist = jax.lax.rsqrt(e2) * e2
    mn = jnp.min(dist, axis=0, keepdims=True)
    kiof = jax.lax.broadcasted_iota(jnp.int32, (_K, 1), 0).astype(jnp.float32)
    idxf = jnp.min(jnp.where(dist == mn, kiof, jnp.float32(2.0**30)), axis=0)
    idx_ref[0, 0] = idxf.astype(jnp.int32)
    onehot = (kiof == idxf[None, :]).astype(jnp.float32)
    z_t = -0.5 * jax.lax.dot_general(cbn, onehot, (((0,), (0,)), ((), ())),
                                     preferred_element_type=jnp.float32)
    zq_ref[0] = xb + (z_t - xb)
    xo_ref[0] = xb


def kernel(x, codebook):
    b, c, l = x.shape
    idx3, zq, xo = pl.pallas_call(
        _vq_tc_body,
        grid=(b,),
        in_specs=[
            pl.BlockSpec((1, c, l), lambda i: (i, 0, 0)),
            pl.BlockSpec((_K, _D), lambda i: (0, 0)),
        ],
        out_specs=[
            pl.BlockSpec((1, 1, l), lambda i: (i, 0, 0)),
            pl.BlockSpec((1, c, l), lambda i: (i, 0, 0)),
            pl.BlockSpec((1, c, l), lambda i: (i, 0, 0)),
        ],
        out_shape=[
            jax.ShapeDtypeStruct((b, 1, l), jnp.int32),
            jax.ShapeDtypeStruct((b, c, l), jnp.float32),
            jax.ShapeDtypeStruct((b, c, l), jnp.float32),
        ],
    )(x, codebook)
    return (zq, xo, idx3.reshape(b, l))
